# baseline (speedup 1.0000x reference)
import jax
import jax.numpy as jnp
from jax import lax
from jax.experimental import pallas as pl
from jax.experimental.pallas import tpu as pltpu
from jax.experimental.pallas import tpu_sc as plsc

D = 64
CHUNK = 128
NBUF = 2
L = 16


def _gather_body(nch, b_per_w, nc,
                 hi_hbm, par_hbm, table_hbm, out_hbm,
                 hi_v, par_v, wides, rows, gsems, osems):
    wid = lax.axis_index("s") * nc + lax.axis_index("c")
    base = wid * b_per_w

    pltpu.sync_copy(hi_hbm.at[wid], hi_v)
    pltpu.sync_copy(par_hbm.at[wid], par_v)

    def gather_start(j, b):
        pltpu.make_async_copy(
            table_hbm.at[hi_v.at[j]], wides[b], gsems[b]).start()

    def gather_wait(j, b):
        pltpu.make_async_copy(
            table_hbm.at[hi_v.at[j]], wides[b], gsems[b]).wait()

    def select(j, b):
        wide, row = wides[b], rows[b]
        lanes = lax.iota(jnp.int32, L)
        for g in range(CHUNK // L):
            rvec = lanes + (g * L)
            pbase = par_v[j, pl.ds(g * L, L)] * D

            def col(c, _):
                cvec = jnp.full((L,), c, jnp.int32)
                v = plsc.load_gather(wide, [rvec, pbase + cvec])
                plsc.store_scatter(row, [rvec, cvec], v)
                return 0

            lax.fori_loop(0, D, col, 0)

    def out_copy(j, b):
        dst = out_hbm.at[pl.ds(base + j * CHUNK, CHUNK)]
        return pltpu.make_async_copy(rows[b], dst, osems[b])

    for b in range(NBUF):
        gather_start(b, b)

    def step(g, carry):
        for b in range(NBUF):
            j = g * NBUF + b
            gather_wait(j, b)
            select(j, b)
            cp = out_copy(j, b)
            cp.start()
            cp.wait()

            @pl.when(j + NBUF < nch)
            def _():
                gather_start(j + NBUF, b)
        return carry

    lax.fori_loop(0, nch // NBUF, step, 0)


def kernel(x, table):
    orig_shape = x.shape
    b = 1
    for s in orig_shape:
        b *= s
    v = table.shape[0]

    info = plsc.get_sparse_core_info()
    nc, ns = info.num_cores, info.num_subcores
    nw = nc * ns
    b_per_w = b // nw
    nch = b_per_w // CHUNK
    assert b == nw * nch * CHUNK and nch % NBUF == 0

    xf = x.reshape(nw, nch, CHUNK).astype(jnp.int32)
    hi = xf >> 1
    par = xf & 1
    th = table.reshape(v // 2, 2 * D)
    mesh = plsc.VectorSubcoreMesh(core_axis_name="c", subcore_axis_name="s")

    scratch = [pltpu.VMEM((nch, CHUNK), jnp.int32),
               pltpu.VMEM((nch, CHUNK), jnp.int32)]
    scratch += [pltpu.VMEM((CHUNK, 2 * D), jnp.float32) for _ in range(NBUF)]
    scratch += [pltpu.VMEM((CHUNK, D), jnp.float32) for _ in range(NBUF)]
    scratch += [pltpu.SemaphoreType.DMA for _ in range(2 * NBUF)]

    def body(hi_hbm, par_hbm, table_hbm, out_hbm, hi_v, par_v, *rest):
        wides = rest[:NBUF]
        rows = rest[NBUF:2 * NBUF]
        gsems = rest[2 * NBUF:3 * NBUF]
        osems = rest[3 * NBUF:]
        _gather_body(nch, b_per_w, nc,
                     hi_hbm, par_hbm, table_hbm, out_hbm,
                     hi_v, par_v, wides, rows, gsems, osems)

    out = pl.kernel(
        body,
        mesh=mesh,
        out_type=jax.ShapeDtypeStruct((b, D), jnp.float32),
        scratch_types=scratch,
        compiler_params=pltpu.CompilerParams(
            use_tc_tiling_on_sc=True, needs_layout_passes=False),
    )(hi, par, th)
    return out.reshape(*orig_shape, D)


# speedup vs baseline: 1.4428x; 1.4428x over previous
import jax
import jax.numpy as jnp
from jax import lax
from jax.experimental import pallas as pl
from jax.experimental.pallas import tpu as pltpu
from jax.experimental.pallas import tpu_sc as plsc

D = 64
CHUNK = 128
NBUF = 2
L = 16


def _gather_body(n_chunks_per_w, bblocks, nc,
                 x_hbm, table_hbm, out_hbm,
                 xv, hiv, wides, trans, gsems, osems):
    wid = lax.axis_index("s") * nc + lax.axis_index("c")
    n_idx = n_chunks_per_w * CHUNK
    base_q = wid * n_chunks_per_w

    pltpu.sync_copy(x_hbm.at[pl.ds(base_q * CHUNK, n_idx)], xv)

    def shift_step(i, carry):
        for u in range(8):
            o = (i * 8 + u) * L
            hiv[pl.ds(o, L)] = lax.shift_right_logical(xv[pl.ds(o, L)], 1)
        return carry

    lax.fori_loop(0, n_idx // (8 * L), shift_step, 0)

    def gather_start(j, b):
        pltpu.make_async_copy(
            table_hbm.at[hiv.at[pl.ds(j * CHUNK, CHUNK)]],
            wides[b], gsems[b]).start()

    def gather_wait(j, b):
        pltpu.make_async_copy(
            table_hbm.at[hiv.at[pl.ds(j * CHUNK, CHUNK)]],
            wides[b], gsems[b]).wait()

    def select_transpose(j, b):
        wide, tr = wides[b], trans[b]
        lanes = lax.iota(jnp.int32, L)
        for g in range(CHUNK // L):
            rvec = lanes + (g * L)
            pbase = (xv[pl.ds(j * CHUNK + g * L, L)] & 1) * D

            def col_step(c4, carry):
                for u in range(L):
                    c = c4 * L + u
                    cvec = jnp.full((L,), c, jnp.int32)
                    v = plsc.load_gather(wide, [rvec, pbase + cvec])
                    tr[c, pl.ds(g * L, L)] = v
                return carry

            lax.fori_loop(0, D // L, col_step, 0)

    def out_copy(j, b):
        q = base_q + j
        t = q // bblocks
        bb = q % bblocks
        dst = out_hbm.at[t, :, pl.ds(bb * CHUNK, CHUNK)]
        return pltpu.make_async_copy(trans[b], dst, osems[b])

    for b in range(NBUF):
        gather_start(b, b)

    def step(g, carry):
        for b in range(NBUF):
            j = g * NBUF + b
            gather_wait(j, b)
            select_transpose(j, b)
            cp = out_copy(j, b)
            cp.start()
            cp.wait()

            @pl.when(j + NBUF < n_chunks_per_w)
            def _():
                gather_start(j + NBUF, b)
        return carry

    lax.fori_loop(0, n_chunks_per_w // NBUF, step, 0)


def kernel(x, table):
    nb, nt = x.shape
    total = nb * nt
    v = table.shape[0]

    info = plsc.get_sparse_core_info()
    nc, ns = info.num_cores, info.num_subcores
    nw = nc * ns
    bblocks = nb // CHUNK
    n_chunks = nt * bblocks
    n_chunks_per_w = n_chunks // nw
    assert n_chunks_per_w * nw == n_chunks and n_chunks_per_w % NBUF == 0
    n_idx = n_chunks_per_w * CHUNK

    xt = x.T.reshape(-1).astype(jnp.int32)
    th = lax.optimization_barrier(table.reshape(-1)).reshape(v // 2, 2 * D)
    mesh = plsc.VectorSubcoreMesh(core_axis_name="c", subcore_axis_name="s")

    scratch = [pltpu.VMEM((n_idx,), jnp.int32),
               pltpu.VMEM((n_idx,), jnp.int32)]
    scratch += [pltpu.VMEM((CHUNK, 2 * D), jnp.float32) for _ in range(NBUF)]
    scratch += [pltpu.VMEM((D, CHUNK), jnp.float32) for _ in range(NBUF)]
    scratch += [pltpu.SemaphoreType.DMA for _ in range(2 * NBUF)]

    def body(x_hbm, table_hbm, out_hbm, xv, hiv, *rest):
        wides = rest[:NBUF]
        trans = rest[NBUF:2 * NBUF]
        gsems = rest[2 * NBUF:3 * NBUF]
        osems = rest[3 * NBUF:]
        _gather_body(n_chunks_per_w, bblocks, nc,
                     x_hbm, table_hbm, out_hbm,
                     xv, hiv, wides, trans, gsems, osems)

    out = pl.kernel(
        body,
        mesh=mesh,
        out_type=jax.ShapeDtypeStruct((nt, D, nb), jnp.float32),
        scratch_types=scratch,
        compiler_params=pltpu.CompilerParams(
            use_tc_tiling_on_sc=True, needs_layout_passes=False),
    )(xt, th)
    return out.transpose(2, 0, 1)


# speedup vs baseline: 2.2547x; 1.5627x over previous
import jax
import jax.numpy as jnp
from jax import lax
from jax.experimental import pallas as pl
from jax.experimental.pallas import tpu as pltpu
from jax.experimental.pallas import tpu_sc as plsc

D = 64
CHUNK = 128
NBUF = 4


def _gather_body(nch, b_per_w, nc,
                 x_hbm, table_hbm, out_hbm,
                 idx_v, rows, gsems, osems):
    wid = lax.axis_index("s") * nc + lax.axis_index("c")
    base = wid * b_per_w

    pltpu.sync_copy(x_hbm.at[wid], idx_v)

    def gather_start(j, b):
        pltpu.make_async_copy(table_hbm.at[idx_v.at[j]], rows[b], gsems[b]).start()

    def gather_wait(j, b):
        pltpu.make_async_copy(table_hbm.at[idx_v.at[j]], rows[b], gsems[b]).wait()

    def out_copy(j, b):
        dst = out_hbm.at[pl.ds(base + j * CHUNK, CHUNK)]
        cp = pltpu.make_async_copy(rows[b], dst, osems[b])
        cp.start()
        return cp

    for b in range(NBUF):
        gather_start(b, b)

    def step(g, carry):
        for b in range(NBUF):
            j = g * NBUF + b
            gather_wait(j, b)
            out_copy(j, b).wait()

            @pl.when(j + NBUF < nch)
            def _():
                gather_start(j + NBUF, b)
        return carry

    lax.fori_loop(0, nch // NBUF, step, 0)


def kernel(x, table):
    orig_shape = x.shape
    b = 1
    for s in orig_shape:
        b *= s

    info = plsc.get_sparse_core_info()
    nc, ns = info.num_cores, info.num_subcores
    nw = nc * ns
    b_per_w = b // nw
    nch = b_per_w // CHUNK
    assert b == nw * nch * CHUNK and nch % NBUF == 0

    xr = x.reshape(nw, nch, CHUNK).astype(jnp.int32)
    mesh = plsc.VectorSubcoreMesh(core_axis_name="c", subcore_axis_name="s")

    scratch = [pltpu.VMEM((nch, CHUNK), jnp.int32)]
    scratch += [pltpu.VMEM((CHUNK, D), jnp.float32) for _ in range(NBUF)]
    scratch += [pltpu.SemaphoreType.DMA for _ in range(2 * NBUF)]

    def body(x_hbm, table_hbm, out_hbm, idx_v, *rest):
        rows = rest[:NBUF]
        gsems = rest[NBUF:2 * NBUF]
        osems = rest[2 * NBUF:]
        _gather_body(nch, b_per_w, nc,
                     x_hbm, table_hbm, out_hbm, idx_v, rows, gsems, osems)

    out = pl.kernel(
        body,
        mesh=mesh,
        out_type=jax.ShapeDtypeStruct((b, D), jnp.float32),
        scratch_types=scratch,
        compiler_params=pltpu.CompilerParams(use_tc_tiling_on_sc=False),
    )(xr, table)
    return out.reshape(*orig_shape, D)


# speedup vs baseline: 3.0689x; 1.3611x over previous
import jax
import jax.numpy as jnp
from jax import lax
from jax.experimental import pallas as pl
from jax.experimental.pallas import tpu as pltpu
from jax.experimental.pallas import tpu_sc as plsc

D = 64
CHUNK = 128
NBUF = 4
TW = 15872


def _transpose_kernel(tt_ref, out_ref):
    t = tt_ref[...].T
    out_ref[:, :D] = t[:TW // 2]
    out_ref[:, D:] = t[TW // 2:]


def _row_major_table(table):
    v, d = table.shape
    tt = table.T
    grid = (v + TW - 1) // TW
    return pl.pallas_call(
        _transpose_kernel,
        grid=(grid,),
        in_specs=[pl.BlockSpec((d, TW), lambda i: (0, i))],
        out_specs=pl.BlockSpec((TW // 2, 2 * d), lambda i: (i, 0)),
        out_shape=jax.ShapeDtypeStruct((grid * TW // 2, 2 * d), jnp.float32),
    )(tt)


def _remap(r):
    i = r // TW
    j = r % TW
    return i * TW + 2 * (j % (TW // 2)) + j // (TW // 2)


def _gather_body(nch, b_per_w, nc,
                 x_hbm, table_hbm, out_hbm,
                 idx_v, rows, gsems, osems):
    wid = lax.axis_index("s") * nc + lax.axis_index("c")
    base = wid * b_per_w

    pltpu.sync_copy(x_hbm.at[wid], idx_v)

    def gather_start(j, b):
        pltpu.make_async_copy(table_hbm.at[idx_v.at[j]], rows[b], gsems[b]).start()

    def gather_wait(j, b):
        pltpu.make_async_copy(table_hbm.at[idx_v.at[j]], rows[b], gsems[b]).wait()

    def out_copy(j, b):
        dst = out_hbm.at[pl.ds(base + j * CHUNK, CHUNK)]
        cp = pltpu.make_async_copy(rows[b], dst, osems[b])
        cp.start()
        return cp

    for b in range(NBUF):
        gather_start(b, b)

    def step(g, carry):
        for b in range(NBUF):
            j = g * NBUF + b
            gather_wait(j, b)
            out_copy(j, b).wait()

            @pl.when(j + NBUF < nch)
            def _():
                gather_start(j + NBUF, b)
        return carry

    lax.fori_loop(0, nch // NBUF, step, 0)


def kernel(x, table):
    orig_shape = x.shape
    b = 1
    for s in orig_shape:
        b *= s

    info = plsc.get_sparse_core_info()
    nc, ns = info.num_cores, info.num_subcores
    nw = nc * ns
    b_per_w = b // nw
    nch = b_per_w // CHUNK
    assert b == nw * nch * CHUNK and nch % NBUF == 0

    xr = _remap(x.reshape(nw, nch, CHUNK).astype(jnp.int32))
    tp = _row_major_table(table)
    tf = tp.reshape(tp.shape[0] * 2, D)
    mesh = plsc.VectorSubcoreMesh(core_axis_name="c", subcore_axis_name="s")

    scratch = [pltpu.VMEM((nch, CHUNK), jnp.int32)]
    scratch += [pltpu.VMEM((CHUNK, D), jnp.float32) for _ in range(NBUF)]
    scratch += [pltpu.SemaphoreType.DMA for _ in range(2 * NBUF)]

    def body(x_hbm, table_hbm, out_hbm, idx_v, *rest):
        rows = rest[:NBUF]
        gsems = rest[NBUF:2 * NBUF]
        osems = rest[2 * NBUF:]
        _gather_body(nch, b_per_w, nc,
                     x_hbm, table_hbm, out_hbm, idx_v, rows, gsems, osems)

    out = pl.kernel(
        body,
        mesh=mesh,
        out_type=jax.ShapeDtypeStruct((b, D), jnp.float32),
        scratch_types=scratch,
        compiler_params=pltpu.CompilerParams(use_tc_tiling_on_sc=False),
    )(xr, tf)
    return out.reshape(*orig_shape, D)


# speedup vs baseline: 3.7054x; 1.2074x over previous
import jax
import jax.numpy as jnp
from jax import lax
from jax.experimental import pallas as pl
from jax.experimental.pallas import tpu as pltpu
from jax.experimental.pallas import tpu_sc as plsc

D = 64
CHUNK = 128
NBUF = 4
TW = 15872


def _transpose_kernel(tt_ref, out_ref):
    t = tt_ref[...].T
    out_ref[:, :D] = t[:TW // 2]
    out_ref[:, D:] = t[TW // 2:]


def _row_major_table(table):
    v, d = table.shape
    tt = table.T
    grid = (v + TW - 1) // TW
    return pl.pallas_call(
        _transpose_kernel,
        grid=(grid,),
        in_specs=[pl.BlockSpec((d, TW), lambda i: (0, i))],
        out_specs=pl.BlockSpec((TW // 2, 2 * d), lambda i: (i, 0)),
        out_shape=jax.ShapeDtypeStruct((grid * TW // 2, 2 * d), jnp.float32),
    )(tt)


def _remap(r):
    i = r // TW
    j = r % TW
    return i * TW + 2 * (j % (TW // 2)) + j // (TW // 2)


def _out_transpose_kernel(in_ref, out_ref):
    out_ref[...] = in_ref[...].T


def _to_batch_minor(flat, nb, nt):
    bw = 128
    out = pl.pallas_call(
        _out_transpose_kernel,
        grid=(nb // bw,),
        in_specs=[pl.BlockSpec((bw, nt * D), lambda i: (i, 0))],
        out_specs=pl.BlockSpec((nt * D, bw), lambda i: (0, i)),
        out_shape=jax.ShapeDtypeStruct((nt * D, nb), jnp.float32),
    )(flat.reshape(nb, nt * D))
    return out.reshape(nt, D, nb).transpose(2, 0, 1)


def _gather_body(nch, b_per_w, nc,
                 x_hbm, table_hbm, out_hbm,
                 idx_v, rows, gsems, osems):
    wid = lax.axis_index("s") * nc + lax.axis_index("c")
    base = wid * b_per_w

    pltpu.sync_copy(x_hbm.at[wid], idx_v)

    def gather_start(j, b):
        pltpu.make_async_copy(table_hbm.at[idx_v.at[j]], rows[b], gsems[b]).start()

    def gather_wait(j, b):
        pltpu.make_async_copy(table_hbm.at[idx_v.at[j]], rows[b], gsems[b]).wait()

    def out_copy(j, b):
        dst = out_hbm.at[pl.ds(base + j * CHUNK, CHUNK)]
        cp = pltpu.make_async_copy(rows[b], dst, osems[b])
        cp.start()
        return cp

    for b in range(NBUF):
        gather_start(b, b)

    def step(g, carry):
        for b in range(NBUF):
            j = g * NBUF + b
            gather_wait(j, b)
            out_copy(j, b).wait()

            @pl.when(j + NBUF < nch)
            def _():
                gather_start(j + NBUF, b)
        return carry

    lax.fori_loop(0, nch // NBUF, step, 0)


def kernel(x, table):
    orig_shape = x.shape
    b = 1
    for s in orig_shape:
        b *= s

    info = plsc.get_sparse_core_info()
    nc, ns = info.num_cores, info.num_subcores
    nw = nc * ns
    b_per_w = b // nw
    nch = b_per_w // CHUNK
    assert b == nw * nch * CHUNK and nch % NBUF == 0

    xr = _remap(x.reshape(nw, nch, CHUNK).astype(jnp.int32))
    tp = _row_major_table(table)
    tf = tp.reshape(tp.shape[0] * 2, D)
    mesh = plsc.VectorSubcoreMesh(core_axis_name="c", subcore_axis_name="s")

    scratch = [pltpu.VMEM((nch, CHUNK), jnp.int32)]
    scratch += [pltpu.VMEM((CHUNK, D), jnp.float32) for _ in range(NBUF)]
    scratch += [pltpu.SemaphoreType.DMA for _ in range(2 * NBUF)]

    def body(x_hbm, table_hbm, out_hbm, idx_v, *rest):
        rows = rest[:NBUF]
        gsems = rest[NBUF:2 * NBUF]
        osems = rest[2 * NBUF:]
        _gather_body(nch, b_per_w, nc,
                     x_hbm, table_hbm, out_hbm, idx_v, rows, gsems, osems)

    out = pl.kernel(
        body,
        mesh=mesh,
        out_type=jax.ShapeDtypeStruct((b, D), jnp.float32),
        scratch_types=scratch,
        compiler_params=pltpu.CompilerParams(use_tc_tiling_on_sc=False),
    )(xr, tf)
    return _to_batch_minor(out, orig_shape[0], orig_shape[1])
